# baseline (speedup 1.0000x reference)
import functools

import jax
import jax.numpy as jnp
from jax import lax
from jax.experimental import pallas as pl
from jax.experimental.pallas import tpu as pltpu
from jax.experimental.pallas import tpu_sc as plsc

NUM_TAG = 100000
D = 128
BATCH = 4096
HIST = 50
N = BATCH * HIST
NC, NS = 2, 16
NW = NC * NS
ROWS_PER_W = N // NW
CHUNK = 640
NCHUNK = ROWS_PER_W // CHUNK


def _sc_body(x_hbm, tags_hbm, table_hbm, out_hbm, idx_v, emb_v, gsem, xsem):
    wid = lax.axis_index("s") * NC + lax.axis_index("c")
    base = wid * ROWS_PER_W

    xcopy = pltpu.make_async_copy(
        x_hbm.at[pl.ds(base, ROWS_PER_W), :],
        out_hbm.at[pl.ds(base, ROWS_PER_W), pl.ds(0, D)],
        xsem,
    )
    xcopy.start()

    def chunk_body(i, _):
        cbase = base + i * CHUNK
        pltpu.sync_copy(tags_hbm.at[pl.ds(cbase, CHUNK)], idx_v)
        pltpu.async_copy(table_hbm.at[idx_v], emb_v, gsem).wait()
        pltpu.sync_copy(emb_v, out_hbm.at[pl.ds(cbase, CHUNK), pl.ds(D, D)])
        return 0

    lax.fori_loop(0, NCHUNK, chunk_body, 0)
    xcopy.wait()


@jax.jit
def _concat_tag(x2d, tags1d, table):
    mesh = plsc.VectorSubcoreMesh(core_axis_name="c", subcore_axis_name="s")
    return pl.kernel(
        _sc_body,
        out_type=jax.ShapeDtypeStruct((N, 2 * D), jnp.float32),
        mesh=mesh,
        scratch_types=[
            pltpu.VMEM((CHUNK,), jnp.int32),
            pltpu.VMEM((CHUNK, D), jnp.float32),
            pltpu.SemaphoreType.DMA,
            pltpu.SemaphoreType.DMA,
        ],
    )(x2d, tags1d, table)


def kernel(x, tags, table):
    x2d = x.reshape(N, D)
    tags1d = tags.reshape(N).astype(jnp.int32)
    out = _concat_tag(x2d, tags1d, table)
    return out.reshape(BATCH, HIST, 2 * D)
